# baseline (speedup 1.0000x reference)
import functools

import jax
import jax.numpy as jnp
from jax.experimental import pallas as pl
from jax.experimental.pallas import tpu as pltpu

B, T, D_WP, HID = 16384, 20, 3, 512
BBc = 512


def _body(wp_ref, wt_ref, pb_ref, emb_ref, out_ref):
    comb = emb_ref[...] + pb_ref[...]
    wpt = [jnp.swapaxes(wp_ref[d], 0, 1) for d in range(D_WP)]
    for t in range(T):
        acc = comb[t : t + 1, :]
        for d in range(D_WP):
            acc = acc + wpt[d][:, t : t + 1] * wt_ref[d : d + 1, :]
        out_ref[t] = acc


@functools.partial(jax.jit)
def kernel(waypoints, proj_w, proj_b, emb_table):
    wpP = waypoints.transpose(2, 1, 0)
    wt = proj_w.T
    pb = proj_b.reshape(1, HID)
    out = pl.pallas_call(
        _body,
        grid=(B // BBc,),
        in_specs=[
            pl.BlockSpec((D_WP, T, BBc), lambda i: (0, 0, i)),
            pl.BlockSpec((D_WP, HID), lambda i: (0, 0)),
            pl.BlockSpec((1, HID), lambda i: (0, 0)),
            pl.BlockSpec((T, HID), lambda i: (0, 0)),
        ],
        out_specs=pl.BlockSpec((T, BBc, HID), lambda i: (0, i, 0)),
        out_shape=jax.ShapeDtypeStruct((T, B, HID), jnp.float32),
        compiler_params=pltpu.CompilerParams(
            dimension_semantics=("parallel",),
        ),
    )(wpP, wt, pb, emb_table)
    return out.transpose(1, 0, 2)


# speedup vs baseline: 1.0138x; 1.0138x over previous
import functools

import jax
import jax.numpy as jnp
from jax.experimental import pallas as pl
from jax.experimental.pallas import tpu as pltpu

B, T, D_WP, HID = 16384, 20, 3, 512
BBig = 4096


def _body(wp_ref, wt_ref, pb_ref, emb_ref, out_ref):
    t = pl.program_id(1)
    acc = emb_ref[0] + pb_ref[...]
    lhs = wp_ref[:, pl.ds(t, 1), :].reshape(D_WP, out_ref.shape[1])
    prod = jax.lax.dot_general(
        lhs,
        wt_ref[...],
        dimension_numbers=(((0,), (0,)), ((), ())),
        preferred_element_type=jnp.float32,
    )
    out_ref[0] = prod + acc


@functools.partial(jax.jit)
def kernel(waypoints, proj_w, proj_b, emb_table):
    wpP = waypoints.transpose(2, 1, 0)
    wt = proj_w.T
    pb = proj_b.reshape(1, HID)
    out = pl.pallas_call(
        _body,
        grid=(B // BBig, T),
        in_specs=[
            pl.BlockSpec((D_WP, T, BBig), lambda i, t: (0, 0, i)),
            pl.BlockSpec((D_WP, HID), lambda i, t: (0, 0)),
            pl.BlockSpec((1, HID), lambda i, t: (0, 0)),
            pl.BlockSpec((1, 1, HID), lambda i, t: (t, 0, 0)),
        ],
        out_specs=pl.BlockSpec((1, BBig, HID), lambda i, t: (t, i, 0)),
        out_shape=jax.ShapeDtypeStruct((T, B, HID), jnp.float32),
        compiler_params=pltpu.CompilerParams(
            dimension_semantics=("arbitrary", "arbitrary"),
        ),
    )(wpP, wt, pb, emb_table.reshape(T, 1, HID))
    return out.transpose(1, 0, 2)


# speedup vs baseline: 1.0217x; 1.0078x over previous
import functools

import jax
import jax.numpy as jnp
from jax.experimental import pallas as pl
from jax.experimental.pallas import tpu as pltpu

B, T, D_WP, HID = 16384, 20, 3, 512
BBig = 8192


def _body(wp_ref, wt_ref, pb_ref, emb_ref, out_ref):
    t = pl.program_id(1)
    acc = emb_ref[0] + pb_ref[...]
    lhs = wp_ref[:, pl.ds(t, 1), :].reshape(D_WP, out_ref.shape[1])
    prod = jax.lax.dot_general(
        lhs,
        wt_ref[...],
        dimension_numbers=(((0,), (0,)), ((), ())),
        preferred_element_type=jnp.float32,
    )
    out_ref[0] = prod + acc


@functools.partial(jax.jit)
def kernel(waypoints, proj_w, proj_b, emb_table):
    wpP = waypoints.transpose(2, 1, 0)
    wt = proj_w.T
    pb = proj_b.reshape(1, HID)
    out = pl.pallas_call(
        _body,
        grid=(B // BBig, T),
        in_specs=[
            pl.BlockSpec((D_WP, T, BBig), lambda i, t: (0, 0, i)),
            pl.BlockSpec((D_WP, HID), lambda i, t: (0, 0)),
            pl.BlockSpec((1, HID), lambda i, t: (0, 0)),
            pl.BlockSpec((1, 1, HID), lambda i, t: (t, 0, 0)),
        ],
        out_specs=pl.BlockSpec((1, BBig, HID), lambda i, t: (t, i, 0)),
        out_shape=jax.ShapeDtypeStruct((T, B, HID), jnp.float32),
        compiler_params=pltpu.CompilerParams(
            dimension_semantics=("arbitrary", "arbitrary"),
        ),
    )(wpP, wt, pb, emb_table.reshape(T, 1, HID))
    return out.transpose(1, 0, 2)


# speedup vs baseline: 1.0273x; 1.0056x over previous
import functools

import jax
import jax.numpy as jnp
from jax.experimental import pallas as pl
from jax.experimental.pallas import tpu as pltpu

B, T, D_WP, HID = 16384, 20, 3, 512
BBig = 8192


def _body(wp_ref, wt_ref, pb_ref, emb_ref, out_ref):
    t = pl.program_id(1)
    acc = emb_ref[pl.ds(t, 1), :] + pb_ref[...]
    lhs = wp_ref[:, pl.ds(t, 1), :].reshape(D_WP, out_ref.shape[1])
    prod = jax.lax.dot_general(
        lhs,
        wt_ref[...],
        dimension_numbers=(((0,), (0,)), ((), ())),
        preferred_element_type=jnp.float32,
    )
    out_ref[0] = prod + acc


@functools.partial(jax.jit)
def kernel(waypoints, proj_w, proj_b, emb_table):
    wpP = waypoints.transpose(2, 1, 0)
    wt = proj_w.T
    pb = proj_b.reshape(1, HID)
    out = pl.pallas_call(
        _body,
        grid=(B // BBig, T),
        in_specs=[
            pl.BlockSpec((D_WP, T, BBig), lambda i, t: (0, 0, i)),
            pl.BlockSpec((D_WP, HID), lambda i, t: (0, 0)),
            pl.BlockSpec((1, HID), lambda i, t: (0, 0)),
            pl.BlockSpec((T, HID), lambda i, t: (0, 0)),
        ],
        out_specs=pl.BlockSpec((1, BBig, HID), lambda i, t: (t, i, 0)),
        out_shape=jax.ShapeDtypeStruct((T, B, HID), jnp.float32),
        compiler_params=pltpu.CompilerParams(
            dimension_semantics=("arbitrary", "arbitrary"),
        ),
    )(wpP, wt, pb, emb_table)
    return out.transpose(1, 0, 2)
